# baseline (speedup 1.0000x reference)
import functools

import jax
import jax.numpy as jnp
from jax import lax
from jax.experimental import pallas as pl
from jax.experimental.pallas import tpu as pltpu
from jax.experimental.pallas import tpu_sc as plsc

BATCH = 16384
K = 16

_info = plsc.get_sparse_core_info()
NC, NS, L = _info.num_cores, _info.num_subcores, _info.num_lanes
NW = NC * NS
BPW = BATCH // NW
CHUNK = 128
NCH = BPW // CHUNK


def _body(users_hbm, items_hbm, gu_hbm, gi_hbm,
          xui_hbm, gu_out_hbm, gi_out_hbm,
          idx_u, idx_i, rows_u, rows_i, xui_v, sem):
    wid = lax.axis_index("s") * NC + lax.axis_index("c")
    base = wid * BPW

    pltpu.sync_copy(users_hbm.at[pl.ds(wid * NCH, NCH)], idx_u)
    pltpu.sync_copy(items_hbm.at[pl.ds(wid * NCH, NCH)], idx_i)

    copies = []
    for j in range(NCH):
        copies.append(pltpu.async_copy(
            gu_hbm.at[idx_u.at[j]], rows_u.at[pl.ds(j * CHUNK, CHUNK)], sem))
        copies.append(pltpu.async_copy(
            gi_hbm.at[idx_i.at[j]], rows_i.at[pl.ds(j * CHUNK, CHUNK)], sem))
    for c in copies:
        c.wait()

    out_u = pltpu.async_copy(rows_u, gu_out_hbm.at[pl.ds(base, BPW)], sem)
    out_i = pltpu.async_copy(rows_i, gi_out_hbm.at[pl.ds(base, BPW)], sem)

    lanes = lax.iota(jnp.int32, L)

    def blk(b, carry):
        row_ids = b * L + lanes
        acc = jnp.zeros((L,), jnp.float32)
        for d in range(K):
            col = jnp.full((L,), d, jnp.int32)
            cu = plsc.load_gather(rows_u, [row_ids, col])
            ci = plsc.load_gather(rows_i, [row_ids, col])
            acc = acc + cu * ci
        xui_v[pl.ds(b * L, L)] = acc
        return carry

    lax.fori_loop(0, BPW // L, blk, 0)

    out_x = pltpu.async_copy(xui_v, xui_hbm.at[pl.ds(base, BPW)], sem)
    out_u.wait()
    out_i.wait()
    out_x.wait()


@jax.jit
def _run(users, items, gu, gi):
    mesh = plsc.VectorSubcoreMesh(core_axis_name="c", subcore_axis_name="s")
    f = pl.kernel(
        _body,
        mesh=mesh,
        compiler_params=pltpu.CompilerParams(
            needs_layout_passes=False, use_tc_tiling_on_sc=False),
        out_type=(
            jax.ShapeDtypeStruct((BATCH,), jnp.float32),
            jax.ShapeDtypeStruct((BATCH, K), jnp.float32),
            jax.ShapeDtypeStruct((BATCH, K), jnp.float32),
        ),
        scratch_types=[
            pltpu.VMEM((NCH, CHUNK), jnp.int32),
            pltpu.VMEM((NCH, CHUNK), jnp.int32),
            pltpu.VMEM((BPW, K), jnp.float32),
            pltpu.VMEM((BPW, K), jnp.float32),
            pltpu.VMEM((BPW,), jnp.float32),
            pltpu.SemaphoreType.DMA,
        ],
    )
    return f(users, items, gu, gi)


def kernel(inputs, Gu, Gi):
    users = inputs[0].reshape(NW * NCH, CHUNK)
    items = inputs[1].reshape(NW * NCH, CHUNK)
    return _run(users, items, Gu, Gi)
